# Initial kernel scaffold; baseline (speedup 1.0000x reference)
import functools

import jax
import jax.numpy as jnp
from jax import lax
from jax.experimental import pallas as pl
from jax.experimental.pallas import tpu as pltpu
from jax.experimental.pallas import tpu_sc as plsc

N = 10000
E = 320000
EL = E + N
F_IN = 128
H1, C1 = 8, 8
D1 = H1 * C1
NUM_CLASSES = 40
D2 = 48

NC, NS, L = 2, 16, 16
CH = 128
N1 = 10240
TRASH = N
N_CHUNKS = 2592
EPAD = N_CHUNKS * CH
CHUNKS_A = N_CHUNKS // NS
CHUNKS_B = N_CHUNKS // (NS * NC)
ROWS_PER_TILE = N1 // NS


def _prep1_body(x_ref, w1_ref, ms_ref, md_ref, h_ref, as_ref, ad_ref):
    h = jnp.dot(x_ref[...], w1_ref[...], preferred_element_type=jnp.float32)
    h_ref[...] = h
    as_ref[...] = jnp.dot(h, ms_ref[...], preferred_element_type=jnp.float32)
    ad_ref[...] = jnp.dot(h, md_ref[...], preferred_element_type=jnp.float32)


def _prep2_body(outp_ref, b1_ref, w2_ref, ms_ref, md_ref, h_ref, as_ref, ad_ref):
    o = outp_ref[0] + outp_ref[1] + b1_ref[...]
    x2 = jnp.where(o > 0, o, jnp.expm1(o))
    h2 = jnp.dot(x2, w2_ref[...], preferred_element_type=jnp.float32)
    h_ref[...] = h2
    as_ref[...] = jnp.dot(h2, ms_ref[...], preferred_element_type=jnp.float32)
    ad_ref[...] = jnp.dot(h2, md_ref[...], preferred_element_type=jnp.float32)


def _final_body(outp_ref, b2_ref, lsm_ref, out_ref):
    o = (outp_ref[0] + outp_ref[1])[:N, :NUM_CLASSES] + b2_ref[...]
    m = jnp.max(o, axis=1, keepdims=True)
    ex = jnp.exp(o - m)
    s = jnp.sum(ex, axis=1, keepdims=True)
    lsm_ref[...] = o - m - jnp.log(s)
    out_ref[...] = o


def _make_sc_layer(D):
    mesh = plsc.VectorSubcoreMesh(core_axis_name="c", subcore_axis_name="s")
    nj = D // L

    def body(src_hbm, dst_hbm, as_hbm, ad_hbm, h_hbm, outp,
             srcv, dstv, asrcv, adstv, sdv, exv, hsv, msgv, s_sh, o_sh, sem):
        c = lax.axis_index("c")
        s = lax.axis_index("s")
        lane = lax.broadcasted_iota(jnp.int32, (L,), 0)
        half = lane // 8
        zero16 = jnp.zeros((L,), jnp.float32)

        def _z16(i, _):
            sdv[i] = zero16
            return 0
        lax.fori_loop(0, CH, _z16, 0)

        def _zmsg(i, _):
            for j in range(nj):
                msgv[i, pl.ds(j * L, L)] = zero16
            return 0
        lax.fori_loop(0, CH, _zmsg, 0)

        for t in range(ROWS_PER_TILE // CH):
            r = s * ROWS_PER_TILE + t * CH
            pltpu.sync_copy(sdv, s_sh.at[pl.ds(r, CH)])
            pltpu.sync_copy(msgv, o_sh.at[pl.ds(r, CH)])
        plsc.subcore_barrier()

        def phase_a(k, _):
            base = (s * CHUNKS_A + k) * CH
            pltpu.sync_copy(src_hbm.at[pl.ds(base, CH)], srcv)
            pltpu.sync_copy(dst_hbm.at[pl.ds(base, CH)], dstv)
            cp1 = pltpu.async_copy(as_hbm.at[srcv], asrcv, sem)
            cp2 = pltpu.async_copy(ad_hbm.at[dstv], adstv, sem)
            cp1.wait()
            cp2.wait()

            def edge(i, _):
                v = asrcv[i] + adstv[i]
                v = jnp.where(v > 0, v, v * 0.2)
                exv[i] = jnp.exp(v)
                return 0
            lax.fori_loop(0, CH, edge, 0)
            pltpu.sync_copy(exv, s_sh.at[dstv], add=True)
            return 0
        lax.fori_loop(0, CHUNKS_A, phase_a, 0)
        plsc.subcore_barrier()

        cjs = [half + 2 * j for j in range(nj)]

        def phase_b(k, _):
            base = ((c * NS + s) * CHUNKS_B + k) * CH
            pltpu.sync_copy(src_hbm.at[pl.ds(base, CH)], srcv)
            pltpu.sync_copy(dst_hbm.at[pl.ds(base, CH)], dstv)
            cp1 = pltpu.async_copy(as_hbm.at[srcv], asrcv, sem)
            cp2 = pltpu.async_copy(ad_hbm.at[dstv], adstv, sem)
            cp3 = pltpu.async_copy(h_hbm.at[srcv], hsv, sem)
            cp4 = pltpu.async_copy(s_sh.at[dstv], sdv, sem)
            cp1.wait()
            cp2.wait()
            cp3.wait()
            cp4.wait()

            def edge(i, _):
                v = asrcv[i] + adstv[i]
                v = jnp.where(v > 0, v, v * 0.2)
                alpha = jnp.exp(v) / (sdv[i] + 1e-16)
                exv[i] = alpha
                row = jnp.full((L,), i, jnp.int32)
                for j in range(nj):
                    aej = plsc.load_gather(exv, [row, cjs[j]])
                    msgv[i, pl.ds(j * L, L)] = hsv[i, pl.ds(j * L, L)] * aej
                return 0
            lax.fori_loop(0, CH, edge, 0)
            pltpu.sync_copy(msgv, o_sh.at[dstv], add=True)
            return 0
        lax.fori_loop(0, CHUNKS_B, phase_b, 0)
        plsc.subcore_barrier()

        for t in range(ROWS_PER_TILE // CH):
            r = s * ROWS_PER_TILE + t * CH
            pltpu.sync_copy(o_sh.at[pl.ds(r, CH)], outp.at[c].at[pl.ds(r, CH)])

    return pl.kernel(
        body,
        out_type=jax.ShapeDtypeStruct((NC, N1, D), jnp.float32),
        mesh=mesh,
        scratch_types=[
            pltpu.VMEM((CH,), jnp.int32),
            pltpu.VMEM((CH,), jnp.int32),
            pltpu.VMEM((CH, L), jnp.float32),
            pltpu.VMEM((CH, L), jnp.float32),
            pltpu.VMEM((CH, L), jnp.float32),
            pltpu.VMEM((CH, L), jnp.float32),
            pltpu.VMEM((CH, D), jnp.float32),
            pltpu.VMEM((CH, D), jnp.float32),
            pltpu.VMEM_SHARED((N1, L), jnp.float32),
            pltpu.VMEM_SHARED((N1, D), jnp.float32),
            pltpu.SemaphoreType.DMA,
        ],
    )


_sc_layer1 = _make_sc_layer(D1)
_sc_layer2 = _make_sc_layer(D2)


def _dup_att_matrix(att_vec, d_in):
    k = jnp.arange(d_in)[:, None]
    l = jnp.arange(L)[None, :]
    if d_in == D1:
        mask = (k // C1) == (l % H1)
    else:
        mask = jnp.ones((d_in, L), bool)
    return att_vec[:, None] * mask.astype(jnp.float32)


def kernel(x, edge_index, W1, att_src1, att_dst1, b1, W2, att_src2, att_dst2, b2):
    ei = edge_index.astype(jnp.int32)
    loops = jnp.arange(N, dtype=jnp.int32)
    padi = jnp.full((EPAD - EL,), TRASH, jnp.int32)
    src = jnp.concatenate([ei[0], loops, padi])
    dst = jnp.concatenate([ei[1], loops, padi])

    xp = jnp.pad(x, ((0, N1 - N), (0, 0)))
    ms1 = _dup_att_matrix(att_src1.reshape(D1), D1)
    md1 = _dup_att_matrix(att_dst1.reshape(D1), D1)
    w2p = jnp.pad(W2, ((0, 0), (0, D2 - NUM_CLASSES)))
    a2s = jnp.pad(att_src2.reshape(NUM_CLASSES), (0, D2 - NUM_CLASSES))
    a2d = jnp.pad(att_dst2.reshape(NUM_CLASSES), (0, D2 - NUM_CLASSES))
    ms2 = _dup_att_matrix(a2s, D2)
    md2 = _dup_att_matrix(a2d, D2)

    h1, as1, ad1 = pl.pallas_call(
        _prep1_body,
        out_shape=[
            jax.ShapeDtypeStruct((N1, D1), jnp.float32),
            jax.ShapeDtypeStruct((N1, L), jnp.float32),
            jax.ShapeDtypeStruct((N1, L), jnp.float32),
        ],
    )(xp, W1, ms1, md1)

    outp1 = _sc_layer1(src, dst, as1, ad1, h1)

    h2, as2, ad2 = pl.pallas_call(
        _prep2_body,
        out_shape=[
            jax.ShapeDtypeStruct((N1, D2), jnp.float32),
            jax.ShapeDtypeStruct((N1, L), jnp.float32),
            jax.ShapeDtypeStruct((N1, L), jnp.float32),
        ],
    )(outp1, b1.reshape(1, D1), w2p, ms2, md2)

    outp2 = _sc_layer2(src, dst, as2, ad2, h2)

    lsm, out = pl.pallas_call(
        _final_body,
        out_shape=[
            jax.ShapeDtypeStruct((N, NUM_CLASSES), jnp.float32),
            jax.ShapeDtypeStruct((N, NUM_CLASSES), jnp.float32),
        ],
    )(outp2, b2.reshape(1, NUM_CLASSES))

    return (lsm, out)


# speedup vs baseline: 25.5107x; 25.5107x over previous
import functools

import jax
import jax.numpy as jnp
from jax import lax
from jax.experimental import pallas as pl
from jax.experimental.pallas import tpu as pltpu
from jax.experimental.pallas import tpu_sc as plsc

N = 10000
E = 320000
EL = E + N
F_IN = 128
H1, C1 = 8, 8
D1 = H1 * C1
NUM_CLASSES = 40
D2 = 48

NC, NS, L = 2, 16, 16
CH = 128
N1 = 10240
TRASH = N
N_CHUNKS = 2592
EPAD = N_CHUNKS * CH
CHUNKS_A = N_CHUNKS // NS
CHUNKS_B = N_CHUNKS // (NS * NC)
ROWS_PER_TILE = N1 // NS


def _prep1_body(x_ref, w1_ref, ms_ref, md_ref, h_ref, as_ref, ad_ref):
    h = jnp.dot(x_ref[...], w1_ref[...], preferred_element_type=jnp.float32)
    h_ref[...] = h
    as_ref[...] = jnp.dot(h, ms_ref[...], preferred_element_type=jnp.float32)
    ad_ref[...] = jnp.dot(h, md_ref[...], preferred_element_type=jnp.float32)


def _prep2_body(outp_ref, b1_ref, w2_ref, ms_ref, md_ref, h_ref, as_ref, ad_ref):
    o = outp_ref[0] + outp_ref[1] + b1_ref[...]
    x2 = jnp.where(o > 0, o, jnp.exp(o) - 1.0)
    h2 = jnp.dot(x2, w2_ref[...], preferred_element_type=jnp.float32)
    h_ref[...] = h2
    as_ref[...] = jnp.dot(h2, ms_ref[...], preferred_element_type=jnp.float32)
    ad_ref[...] = jnp.dot(h2, md_ref[...], preferred_element_type=jnp.float32)


def _final_body(outp_ref, b2_ref, lsm_ref, out_ref):
    o = (outp_ref[0] + outp_ref[1])[:N, :NUM_CLASSES] + b2_ref[...]
    m = jnp.max(o, axis=1, keepdims=True)
    ex = jnp.exp(o - m)
    s = jnp.sum(ex, axis=1, keepdims=True)
    lsm_ref[...] = o - m - jnp.log(s)
    out_ref[...] = o


def _make_sc_layer(D):
    mesh = plsc.VectorSubcoreMesh(core_axis_name="c", subcore_axis_name="s")
    nj = D // L

    def body(src_hbm, dst_hbm, as_hbm, ad_hbm, h_hbm, outp,
             srcv, dstv, asrcv, adstv, sdv, exv, alf, hsv, msgv, s_sh, o_sh, sem):
        c = lax.axis_index("c")
        s = lax.axis_index("s")
        lane = lax.broadcasted_iota(jnp.int32, (L,), 0)
        half = lane // 8
        zero16 = jnp.zeros((L,), jnp.float32)

        def _z16(i, _):
            sdv[i] = zero16
            return 0
        lax.fori_loop(0, CH, _z16, 0)

        def _zmsg(i, _):
            for j in range(nj):
                msgv[i, pl.ds(j * L, L)] = zero16
            return 0
        lax.fori_loop(0, CH, _zmsg, 0)

        for t in range(ROWS_PER_TILE // CH):
            r = s * ROWS_PER_TILE + t * CH
            pltpu.sync_copy(sdv, s_sh.at[pl.ds(r, CH)])
            pltpu.sync_copy(msgv, o_sh.at[pl.ds(r, CH)])
        plsc.subcore_barrier()

        def phase_a(k, _):
            base = (s * CHUNKS_A + k) * CH
            pltpu.sync_copy(src_hbm.at[pl.ds(base, CH)], srcv)
            pltpu.sync_copy(dst_hbm.at[pl.ds(base, CH)], dstv)
            cp1 = pltpu.async_copy(as_hbm.at[srcv], asrcv, sem)
            cp2 = pltpu.async_copy(ad_hbm.at[dstv], adstv, sem)
            cp1.wait()
            cp2.wait()

            def edge(i, _):
                v = asrcv[i] + adstv[i]
                v = jnp.where(v > 0, v, v * 0.2)
                exv[i] = jnp.exp(v)
                return 0
            lax.fori_loop(0, CH, edge, 0)
            pltpu.sync_copy(exv, s_sh.at[dstv], add=True)
            return 0
        lax.fori_loop(0, CHUNKS_A, phase_a, 0)
        plsc.subcore_barrier()


        def phase_b(k, _):
            base = ((c * NS + s) * CHUNKS_B + k) * CH
            pltpu.sync_copy(src_hbm.at[pl.ds(base, CH)], srcv)
            pltpu.sync_copy(dst_hbm.at[pl.ds(base, CH)], dstv)
            cp1 = pltpu.async_copy(as_hbm.at[srcv], asrcv, sem)
            cp2 = pltpu.async_copy(ad_hbm.at[dstv], adstv, sem)
            cp3 = pltpu.async_copy(h_hbm.at[srcv], hsv, sem)
            cp1.wait()
            cp2.wait()
            cp3.wait()
            cp4 = pltpu.async_copy(s_sh.at[dstv], sdv, sem)
            cp4.wait()

            def edge(i, _):
                v = asrcv[i] + adstv[i]
                v = jnp.where(v > 0, v, v * 0.2)
                alpha = jnp.exp(v) / (sdv[i] + 1e-16)
                alf[pl.ds(i * L, L)] = alpha
                lane_i = lax.broadcasted_iota(jnp.int32, (L,), 0)
                half_i = lane_i // 8
                for j in range(nj):
                    cj = half_i + 2 * j
                    aej = plsc.load_gather(alf, [i * L + cj])
                    msgv[i, pl.ds(j * L, L)] = hsv[i, pl.ds(j * L, L)] * aej
                return 0
            lax.fori_loop(0, CH, edge, 0)
            pltpu.sync_copy(msgv, o_sh.at[dstv], add=True)
            return 0
        lax.fori_loop(0, CHUNKS_B, phase_b, 0)
        plsc.subcore_barrier()

        for t in range(ROWS_PER_TILE // CH):
            r = s * ROWS_PER_TILE + t * CH
            pltpu.sync_copy(o_sh.at[pl.ds(r, CH)], outp.at[c].at[pl.ds(r, CH)])

    return pl.kernel(
        body,
        out_type=jax.ShapeDtypeStruct((NC, N1, D), jnp.float32),
        mesh=mesh,
        scratch_types=[
            pltpu.VMEM((CH,), jnp.int32),
            pltpu.VMEM((CH,), jnp.int32),
            pltpu.VMEM((CH, L), jnp.float32),
            pltpu.VMEM((CH, L), jnp.float32),
            pltpu.VMEM((CH, L), jnp.float32),
            pltpu.VMEM((CH, L), jnp.float32),
            pltpu.VMEM((CH * L,), jnp.float32),
            pltpu.VMEM((CH, D), jnp.float32),
            pltpu.VMEM((CH, D), jnp.float32),
            pltpu.VMEM_SHARED((N1, L), jnp.float32),
            pltpu.VMEM_SHARED((N1, D), jnp.float32),
            pltpu.SemaphoreType.DMA,
        ],
        compiler_params=pltpu.CompilerParams(use_tc_tiling_on_sc=False, needs_layout_passes=False),
    )


_sc_layer1 = _make_sc_layer(D1)
_sc_layer2 = _make_sc_layer(D2)


def _dup_att_matrix(att_vec, d_in):
    k = jnp.arange(d_in)[:, None]
    l = jnp.arange(L)[None, :]
    if d_in == D1:
        mask = (k // C1) == (l % H1)
    else:
        mask = jnp.ones((d_in, L), bool)
    return att_vec[:, None] * mask.astype(jnp.float32)


def kernel(x, edge_index, W1, att_src1, att_dst1, b1, W2, att_src2, att_dst2, b2):
    ei = edge_index.astype(jnp.int32)
    loops = jnp.arange(N, dtype=jnp.int32)
    padi = jnp.full((EPAD - EL,), TRASH, jnp.int32)
    src = jnp.concatenate([ei[0], loops, padi])
    dst = jnp.concatenate([ei[1], loops, padi])

    xp = jnp.pad(x, ((0, N1 - N), (0, 0)))
    ms1 = _dup_att_matrix(att_src1.reshape(D1), D1)
    md1 = _dup_att_matrix(att_dst1.reshape(D1), D1)
    w2p = jnp.pad(W2, ((0, 0), (0, D2 - NUM_CLASSES)))
    a2s = jnp.pad(att_src2.reshape(NUM_CLASSES), (0, D2 - NUM_CLASSES))
    a2d = jnp.pad(att_dst2.reshape(NUM_CLASSES), (0, D2 - NUM_CLASSES))
    ms2 = _dup_att_matrix(a2s, D2)
    md2 = _dup_att_matrix(a2d, D2)

    h1, as1, ad1 = pl.pallas_call(
        _prep1_body,
        out_shape=[
            jax.ShapeDtypeStruct((N1, D1), jnp.float32),
            jax.ShapeDtypeStruct((N1, L), jnp.float32),
            jax.ShapeDtypeStruct((N1, L), jnp.float32),
        ],
    )(xp, W1, ms1, md1)

    outp1 = _sc_layer1(src, dst, as1, ad1, h1)

    h2, as2, ad2 = pl.pallas_call(
        _prep2_body,
        out_shape=[
            jax.ShapeDtypeStruct((N1, D2), jnp.float32),
            jax.ShapeDtypeStruct((N1, L), jnp.float32),
            jax.ShapeDtypeStruct((N1, L), jnp.float32),
        ],
    )(outp1, b1.reshape(1, D1), w2p, ms2, md2)

    outp2 = _sc_layer2(src, dst, as2, ad2, h2)

    lsm, out = pl.pallas_call(
        _final_body,
        out_shape=[
            jax.ShapeDtypeStruct((N, NUM_CLASSES), jnp.float32),
            jax.ShapeDtypeStruct((N, NUM_CLASSES), jnp.float32),
        ],
    )(outp2, b2.reshape(1, NUM_CLASSES))

    return (lsm, out)


# speedup vs baseline: 41.3036x; 1.6191x over previous
import jax
import jax.numpy as jnp
from jax import lax
from jax.experimental import pallas as pl
from jax.experimental.pallas import tpu as pltpu
from jax.experimental.pallas import tpu_sc as plsc

N = 10000
E = 320000
EL = E + N
F_IN = 128
H1, C1 = 8, 8
D1 = H1 * C1
NUM_CLASSES = 40
D2 = 48

NC, NS, L = 2, 16, 16
CH = 128
N1 = 10240
TRASH = N
N_CHUNKS = 2592
EPAD = N_CHUNKS * CH
CHUNKS_W = N_CHUNKS // (NS * NC)
ROWS_PER_TILE = N1 // NS


def _prep1_body(x_ref, w1_ref, ms_ref, md_ref, has_ref, ad_ref):
    h = jnp.dot(x_ref[...], w1_ref[...], preferred_element_type=jnp.float32)
    hs = jnp.dot(h, ms_ref[...], preferred_element_type=jnp.float32)
    has_ref[...] = jnp.concatenate([h, hs], axis=1)
    ad_ref[...] = jnp.dot(h, md_ref[...], preferred_element_type=jnp.float32)


def _prep2_body(acc_ref, b1_ref, rsel_ref, w2_ref, ms_ref, md_ref,
                has_ref, ad_ref):
    o = acc_ref[0] + acc_ref[1]
    num = o[:, :D1]
    srec = 1.0 / (o[:, D1:D1 + H1] + 1e-16)
    sexp = jnp.dot(srec, rsel_ref[...], preferred_element_type=jnp.float32)
    o1 = num * sexp + b1_ref[...]
    x2 = jnp.where(o1 > 0, o1, jnp.exp(o1) - 1.0)
    h2 = jnp.dot(x2, w2_ref[...], preferred_element_type=jnp.float32)
    hs = jnp.dot(h2, ms_ref[...], preferred_element_type=jnp.float32)
    has_ref[...] = jnp.concatenate([h2, hs], axis=1)
    ad_ref[...] = jnp.dot(h2, md_ref[...], preferred_element_type=jnp.float32)


def _final_body(acc_ref, b2_ref, lsm_ref, out_ref):
    o = acc_ref[0] + acc_ref[1]
    num = o[:N, :NUM_CLASSES]
    sden = o[:N, D2:D2 + 1]
    res = num / (sden + 1e-16) + b2_ref[...]
    m = jnp.max(res, axis=1, keepdims=True)
    ex = jnp.exp(res - m)
    ssum = jnp.sum(ex, axis=1, keepdims=True)
    lsm_ref[...] = res - m - jnp.log(ssum)
    out_ref[...] = res


def _make_sc_layer(DH):
    DT = DH + L
    mesh = plsc.VectorSubcoreMesh(core_axis_name="c", subcore_axis_name="s")
    nj = DH // L

    def body(src_hbm, dst_hbm, has_hbm, ad_hbm, outp,
             srcv, dstv, hasv, adv, alf, msgv, acc_sh, sem):
        c = lax.axis_index("c")
        s = lax.axis_index("s")
        zero16 = jnp.zeros((L,), jnp.float32)

        def _zmsg(i, _):
            for j in range(DT // L):
                msgv[i, pl.ds(j * L, L)] = zero16
            return 0
        lax.fori_loop(0, CH, _zmsg, 0)

        for t in range(ROWS_PER_TILE // CH):
            r = s * ROWS_PER_TILE + t * CH
            pltpu.sync_copy(msgv, acc_sh.at[pl.ds(r, CH)])
        plsc.subcore_barrier()

        def chunk_body(k, _):
            base = ((c * NS + s) * CHUNKS_W + k) * CH
            pltpu.sync_copy(src_hbm.at[pl.ds(base, CH)], srcv)
            pltpu.sync_copy(dst_hbm.at[pl.ds(base, CH)], dstv)
            cp1 = pltpu.async_copy(has_hbm.at[srcv], hasv, sem)
            cp2 = pltpu.async_copy(ad_hbm.at[dstv], adv, sem)
            cp1.wait()
            cp2.wait()

            def edge(i, _):
                v = hasv[i, pl.ds(DH, L)] + adv[i]
                v = jnp.where(v > 0, v, v * 0.2)
                ex = jnp.exp(v)
                alf[pl.ds(i * L, L)] = ex
                msgv[i, pl.ds(DH, L)] = ex
                lane = lax.broadcasted_iota(jnp.int32, (L,), 0)
                half = lane // 8
                for j in range(nj):
                    aej = plsc.load_gather(alf, [i * L + half + 2 * j])
                    msgv[i, pl.ds(j * L, L)] = hasv[i, pl.ds(j * L, L)] * aej
                return 0
            lax.fori_loop(0, CH, edge, 0)
            pltpu.sync_copy(msgv, acc_sh.at[dstv], add=True)
            return 0
        lax.fori_loop(0, CHUNKS_W, chunk_body, 0)
        plsc.subcore_barrier()

        for t in range(ROWS_PER_TILE // CH):
            r = s * ROWS_PER_TILE + t * CH
            pltpu.sync_copy(acc_sh.at[pl.ds(r, CH)], outp.at[c].at[pl.ds(r, CH)])

    return pl.kernel(
        body,
        out_type=jax.ShapeDtypeStruct((NC, N1, DT), jnp.float32),
        mesh=mesh,
        scratch_types=[
            pltpu.VMEM((CH,), jnp.int32),
            pltpu.VMEM((CH,), jnp.int32),
            pltpu.VMEM((CH, DT), jnp.float32),
            pltpu.VMEM((CH, L), jnp.float32),
            pltpu.VMEM((CH * L,), jnp.float32),
            pltpu.VMEM((CH, DT), jnp.float32),
            pltpu.VMEM_SHARED((N1, DT), jnp.float32),
            pltpu.SemaphoreType.DMA,
        ],
        compiler_params=pltpu.CompilerParams(
            use_tc_tiling_on_sc=False, needs_layout_passes=False),
    )


_sc_layer1 = _make_sc_layer(D1)
_sc_layer2 = _make_sc_layer(D2)


def _dup_att_matrix(att_vec, d_in):
    k = jnp.arange(d_in)[:, None]
    l = jnp.arange(L)[None, :]
    if d_in == D1:
        mask = (k // C1) == (l % H1)
    else:
        mask = jnp.ones((d_in, L), bool)
    return att_vec[:, None] * mask.astype(jnp.float32)


def kernel(x, edge_index, W1, att_src1, att_dst1, b1, W2, att_src2, att_dst2, b2):
    ei = edge_index.astype(jnp.int32)
    loops = jnp.arange(N, dtype=jnp.int32)
    padi = jnp.full((EPAD - EL,), TRASH, jnp.int32)
    src = jnp.concatenate([ei[0], loops, padi])
    dst = jnp.concatenate([ei[1], loops, padi])

    xp = jnp.pad(x, ((0, N1 - N), (0, 0)))
    ms1 = _dup_att_matrix(att_src1.reshape(D1), D1)
    md1 = _dup_att_matrix(att_dst1.reshape(D1), D1)
    w2p = jnp.pad(W2, ((0, 0), (0, D2 - NUM_CLASSES)))
    a2s = jnp.pad(att_src2.reshape(NUM_CLASSES), (0, D2 - NUM_CLASSES))
    a2d = jnp.pad(att_dst2.reshape(NUM_CLASSES), (0, D2 - NUM_CLASSES))
    ms2 = _dup_att_matrix(a2s, D2)
    md2 = _dup_att_matrix(a2d, D2)
    rsel = (jnp.arange(D1)[None, :] // C1 == jnp.arange(H1)[:, None])
    rsel = rsel.astype(jnp.float32)

    has1, ad1 = pl.pallas_call(
        _prep1_body,
        out_shape=[
            jax.ShapeDtypeStruct((N1, D1 + L), jnp.float32),
            jax.ShapeDtypeStruct((N1, L), jnp.float32),
        ],
    )(xp, W1, ms1, md1)

    acc1 = _sc_layer1(src, dst, has1, ad1)

    has2, ad2 = pl.pallas_call(
        _prep2_body,
        out_shape=[
            jax.ShapeDtypeStruct((N1, D2 + L), jnp.float32),
            jax.ShapeDtypeStruct((N1, L), jnp.float32),
        ],
    )(acc1, b1.reshape(1, D1), rsel, w2p, ms2, md2)

    acc2 = _sc_layer2(src, dst, has2, ad2)

    lsm, out = pl.pallas_call(
        _final_body,
        out_shape=[
            jax.ShapeDtypeStruct((N, NUM_CLASSES), jnp.float32),
            jax.ShapeDtypeStruct((N, NUM_CLASSES), jnp.float32),
        ],
    )(acc2, b2.reshape(1, NUM_CLASSES))

    return (lsm, out)


# speedup vs baseline: 41.9012x; 1.0145x over previous
import jax
import jax.numpy as jnp
from jax import lax
from jax.experimental import pallas as pl
from jax.experimental.pallas import tpu as pltpu
from jax.experimental.pallas import tpu_sc as plsc

N = 10000
E = 320000
EL = E + N
F_IN = 128
H1, C1 = 8, 8
D1 = H1 * C1
NUM_CLASSES = 40
D2 = 48

NC, NS, L = 2, 16, 16
CH = 128
N1 = 10240
TRASH = N
N_CHUNKS = 2592
EPAD = N_CHUNKS * CH
CHUNKS_W = N_CHUNKS // (NS * NC)
ROWS_PER_TILE = N1 // NS
UNROLL = 4


def _prep1_body(x_ref, w1_ref, ms_ref, md_ref, has_ref, ad_ref):
    h = jnp.dot(x_ref[...], w1_ref[...], preferred_element_type=jnp.float32)
    hs = jnp.dot(h, ms_ref[...], preferred_element_type=jnp.float32)
    has_ref[...] = jnp.concatenate([h, hs], axis=1)
    ad_ref[...] = jnp.dot(h, md_ref[...], preferred_element_type=jnp.float32)


def _prep2_body(acc_ref, b1_ref, rsel_ref, w2_ref, ms_ref, md_ref,
                has_ref, ad_ref):
    o = acc_ref[0] + acc_ref[1]
    num = o[:, :D1]
    srec = 1.0 / (o[:, D1:D1 + H1] + 1e-16)
    sexp = jnp.dot(srec, rsel_ref[...], preferred_element_type=jnp.float32)
    o1 = num * sexp + b1_ref[...]
    x2 = jnp.where(o1 > 0, o1, jnp.exp(o1) - 1.0)
    h2 = jnp.dot(x2, w2_ref[...], preferred_element_type=jnp.float32)
    hs = jnp.dot(h2, ms_ref[...], preferred_element_type=jnp.float32)
    has_ref[...] = jnp.concatenate([h2, hs], axis=1)
    ad_ref[...] = jnp.dot(h2, md_ref[...], preferred_element_type=jnp.float32)


def _final_body(acc_ref, b2_ref, lsm_ref, out_ref):
    o = acc_ref[0] + acc_ref[1]
    num = o[:N, :NUM_CLASSES]
    sden = o[:N, D2:D2 + 1]
    res = num / (sden + 1e-16) + b2_ref[...]
    m = jnp.max(res, axis=1, keepdims=True)
    ex = jnp.exp(res - m)
    ssum = jnp.sum(ex, axis=1, keepdims=True)
    lsm_ref[...] = res - m - jnp.log(ssum)
    out_ref[...] = res


def _make_sc_layer(DH):
    DT = DH + L
    mesh = plsc.VectorSubcoreMesh(core_axis_name="c", subcore_axis_name="s")
    nj = DH // L

    def body(src_hbm, dst_hbm, has_hbm, ad_hbm, outp,
             srcv, dstv, hasv, adv, alf, msgv, acc_sh, sem):
        c = lax.axis_index("c")
        s = lax.axis_index("s")
        zero16 = jnp.zeros((L,), jnp.float32)

        def _zmsg(i, _):
            for j in range(DT // L):
                msgv[i, pl.ds(j * L, L)] = zero16
            return 0
        lax.fori_loop(0, CH, _zmsg, 0)

        for t in range(ROWS_PER_TILE // CH):
            r = s * ROWS_PER_TILE + t * CH
            pltpu.sync_copy(msgv, acc_sh.at[pl.ds(r, CH)])
        plsc.subcore_barrier()

        def chunk_body(k, _):
            base = ((c * NS + s) * CHUNKS_W + k) * CH
            pltpu.sync_copy(src_hbm.at[pl.ds(base, CH)], srcv)
            pltpu.sync_copy(dst_hbm.at[pl.ds(base, CH)], dstv)
            cp1 = pltpu.async_copy(has_hbm.at[srcv], hasv, sem)
            cp2 = pltpu.async_copy(ad_hbm.at[dstv], adv, sem)
            cp1.wait()
            cp2.wait()

            def group(g, _):
                i0 = g * UNROLL
                for u in range(UNROLL):
                    i = i0 + u
                    v = hasv[i, pl.ds(DH, L)] + adv[i]
                    v = jnp.maximum(v, v * 0.2)
                    ex = jnp.exp(v)
                    alf[pl.ds(u * L, L)] = ex
                    msgv[i, pl.ds(DH, L)] = ex
                lane = lax.broadcasted_iota(jnp.int32, (L,), 0)
                half = lane // 8
                for u in range(UNROLL):
                    i = i0 + u
                    for j in range(nj):
                        aej = plsc.load_gather(alf, [half + (u * L + 2 * j)])
                        msgv[i, pl.ds(j * L, L)] = (
                            hasv[i, pl.ds(j * L, L)] * aej)
                return 0
            lax.fori_loop(0, CH // UNROLL, group, 0)
            pltpu.sync_copy(msgv, acc_sh.at[dstv], add=True)
            return 0
        lax.fori_loop(0, CHUNKS_W, chunk_body, 0)
        plsc.subcore_barrier()

        for t in range(ROWS_PER_TILE // CH):
            r = s * ROWS_PER_TILE + t * CH
            pltpu.sync_copy(acc_sh.at[pl.ds(r, CH)], outp.at[c].at[pl.ds(r, CH)])

    return pl.kernel(
        body,
        out_type=jax.ShapeDtypeStruct((NC, N1, DT), jnp.float32),
        mesh=mesh,
        scratch_types=[
            pltpu.VMEM((CH,), jnp.int32),
            pltpu.VMEM((CH,), jnp.int32),
            pltpu.VMEM((CH, DT), jnp.float32),
            pltpu.VMEM((CH, L), jnp.float32),
            pltpu.VMEM((CH * L,), jnp.float32),
            pltpu.VMEM((CH, DT), jnp.float32),
            pltpu.VMEM_SHARED((N1, DT), jnp.float32),
            pltpu.SemaphoreType.DMA,
        ],
        compiler_params=pltpu.CompilerParams(
            use_tc_tiling_on_sc=False, needs_layout_passes=False),
    )


_sc_layer1 = _make_sc_layer(D1)
_sc_layer2 = _make_sc_layer(D2)


def _dup_att_matrix(att_vec, d_in):
    k = jnp.arange(d_in)[:, None]
    l = jnp.arange(L)[None, :]
    if d_in == D1:
        mask = (k // C1) == (l % H1)
    else:
        mask = jnp.ones((d_in, L), bool)
    return att_vec[:, None] * mask.astype(jnp.float32)


def kernel(x, edge_index, W1, att_src1, att_dst1, b1, W2, att_src2, att_dst2, b2):
    ei = edge_index.astype(jnp.int32)
    loops = jnp.arange(N, dtype=jnp.int32)
    padi = jnp.full((EPAD - EL,), TRASH, jnp.int32)
    src = jnp.concatenate([ei[0], loops, padi])
    dst = jnp.concatenate([ei[1], loops, padi])

    xp = jnp.pad(x, ((0, N1 - N), (0, 0)))
    ms1 = _dup_att_matrix(att_src1.reshape(D1), D1)
    md1 = _dup_att_matrix(att_dst1.reshape(D1), D1)
    w2p = jnp.pad(W2, ((0, 0), (0, D2 - NUM_CLASSES)))
    a2s = jnp.pad(att_src2.reshape(NUM_CLASSES), (0, D2 - NUM_CLASSES))
    a2d = jnp.pad(att_dst2.reshape(NUM_CLASSES), (0, D2 - NUM_CLASSES))
    ms2 = _dup_att_matrix(a2s, D2)
    md2 = _dup_att_matrix(a2d, D2)
    rsel = (jnp.arange(D1)[None, :] // C1 == jnp.arange(H1)[:, None])
    rsel = rsel.astype(jnp.float32)

    has1, ad1 = pl.pallas_call(
        _prep1_body,
        out_shape=[
            jax.ShapeDtypeStruct((N1, D1 + L), jnp.float32),
            jax.ShapeDtypeStruct((N1, L), jnp.float32),
        ],
    )(xp, W1, ms1, md1)

    acc1 = _sc_layer1(src, dst, has1, ad1)

    has2, ad2 = pl.pallas_call(
        _prep2_body,
        out_shape=[
            jax.ShapeDtypeStruct((N1, D2 + L), jnp.float32),
            jax.ShapeDtypeStruct((N1, L), jnp.float32),
        ],
    )(acc1, b1.reshape(1, D1), rsel, w2p, ms2, md2)

    acc2 = _sc_layer2(src, dst, has2, ad2)

    lsm, out = pl.pallas_call(
        _final_body,
        out_shape=[
            jax.ShapeDtypeStruct((N, NUM_CLASSES), jnp.float32),
            jax.ShapeDtypeStruct((N, NUM_CLASSES), jnp.float32),
        ],
    )(acc2, b2.reshape(1, NUM_CLASSES))

    return (lsm, out)


# speedup vs baseline: 56.9495x; 1.3591x over previous
import jax
import jax.numpy as jnp
from jax import lax
from jax.experimental import pallas as pl
from jax.experimental.pallas import tpu as pltpu
from jax.experimental.pallas import tpu_sc as plsc

N = 10000
E = 320000
EL = E + N
F_IN = 128
H1, C1 = 8, 8
D1 = H1 * C1
NUM_CLASSES = 40
D2 = 48

NC, NS, L = 2, 16, 16
CH = 128
N1 = 10240
TRASH = N
N_CHUNKS = 2592
EPAD = N_CHUNKS * CH
CHUNKS_W = N_CHUNKS // (NS * NC)
ROWS_PER_TILE = N1 // NS
UNROLL = 4


def _prep1_body(x_ref, w1_ref, ms_ref, md_ref, has_ref, ad_ref):
    h = jnp.dot(x_ref[...], w1_ref[...], preferred_element_type=jnp.float32)
    hs = jnp.dot(h, ms_ref[...], preferred_element_type=jnp.float32)
    has_ref[...] = jnp.concatenate([h, hs], axis=1)
    ad_ref[...] = jnp.dot(h, md_ref[...], preferred_element_type=jnp.float32)


def _prep2_body(acc_ref, b1_ref, rsel_ref, w2_ref, ms_ref, md_ref,
                has_ref, ad_ref):
    o = acc_ref[0] + acc_ref[1]
    num = o[:, :D1]
    srec = 1.0 / (o[:, D1:D1 + H1] + 1e-16)
    sexp = jnp.dot(srec, rsel_ref[...], preferred_element_type=jnp.float32)
    o1 = num * sexp + b1_ref[...]
    x2 = jnp.where(o1 > 0, o1, jnp.exp(o1) - 1.0)
    h2 = jnp.dot(x2, w2_ref[...], preferred_element_type=jnp.float32)
    hs = jnp.dot(h2, ms_ref[...], preferred_element_type=jnp.float32)
    has_ref[...] = jnp.concatenate([h2, hs], axis=1)
    ad_ref[...] = jnp.dot(h2, md_ref[...], preferred_element_type=jnp.float32)


def _final_body(acc_ref, b2_ref, lsm_ref, out_ref):
    o = acc_ref[0] + acc_ref[1]
    num = o[:N, :NUM_CLASSES]
    sden = o[:N, D2:D2 + 1]
    res = num / (sden + 1e-16) + b2_ref[...]
    m = jnp.max(res, axis=1, keepdims=True)
    ex = jnp.exp(res - m)
    ssum = jnp.sum(ex, axis=1, keepdims=True)
    lsm_ref[...] = res - m - jnp.log(ssum)
    out_ref[...] = res


def _make_sc_layer(DH):
    DT = DH + L
    mesh = plsc.VectorSubcoreMesh(core_axis_name="c", subcore_axis_name="s")
    nj = DH // L

    assert CHUNKS_W % 2 == 1

    def body(ei_hbm, has_hbm, ad_hbm, outp,
             idx0, idx1, has0, has1, ad0, ad1, alf, msg0, msg1, acc_sh,
             semg0, semg1, sem):
        c = lax.axis_index("c")
        s = lax.axis_index("s")
        wid = c * NS + s
        zero16 = jnp.zeros((L,), jnp.float32)
        idxv = [idx0, idx1]
        hasv = [has0, has1]
        adv = [ad0, ad1]
        msgv = [msg0, msg1]
        semg = [semg0, semg1]

        def _zmsg(i, _):
            for j in range(DT // L):
                msg0[i, pl.ds(j * L, L)] = zero16
            return 0
        lax.fori_loop(0, CH, _zmsg, 0)

        for t in range(ROWS_PER_TILE // CH):
            r = s * ROWS_PER_TILE + t * CH
            pltpu.sync_copy(msg0, acc_sh.at[pl.ds(r, CH)])
        plsc.subcore_barrier()

        def start_load(b, k):
            pltpu.sync_copy(ei_hbm.at[wid * CHUNKS_W + k], idxv[b])
            pltpu.async_copy(has_hbm.at[idxv[b].at[0]], hasv[b], semg[b])
            pltpu.async_copy(ad_hbm.at[idxv[b].at[1]], adv[b], semg[b])

        def wait_load(b):
            pltpu.make_async_copy(has_hbm.at[idxv[b].at[0]], hasv[b],
                                  semg[b]).wait()
            pltpu.make_async_copy(ad_hbm.at[idxv[b].at[1]], adv[b],
                                  semg[b]).wait()

        def compute(b):
            def group(g, _):
                i0 = g * UNROLL
                for u in range(UNROLL):
                    i = i0 + u
                    v = hasv[b][i, pl.ds(DH, L)] + adv[b][i]
                    v = jnp.maximum(v, v * 0.2)
                    ex = jnp.exp(v)
                    alf[pl.ds(u * L, L)] = ex
                    msgv[b][i, pl.ds(DH, L)] = ex
                lane = lax.broadcasted_iota(jnp.int32, (L,), 0)
                half = lane // 8
                for u in range(UNROLL):
                    i = i0 + u
                    for j in range(nj):
                        aej = plsc.load_gather(alf, [half + (u * L + 2 * j)])
                        msgv[b][i, pl.ds(j * L, L)] = (
                            hasv[b][i, pl.ds(j * L, L)] * aej)
                return 0
            lax.fori_loop(0, CH // UNROLL, group, 0)
            pltpu.sync_copy(msgv[b], acc_sh.at[idxv[b].at[1]], add=True)

        start_load(0, 0)

        def pair(p, _):
            k = 2 * p
            start_load(1, k + 1)
            wait_load(0)
            compute(0)
            start_load(0, k + 2)
            wait_load(1)
            compute(1)
            return 0
        lax.fori_loop(0, (CHUNKS_W - 1) // 2, pair, 0)
        wait_load(0)
        compute(0)
        plsc.subcore_barrier()

        for t in range(ROWS_PER_TILE // CH):
            r = s * ROWS_PER_TILE + t * CH
            pltpu.sync_copy(acc_sh.at[pl.ds(r, CH)], outp.at[c].at[pl.ds(r, CH)])

    return pl.kernel(
        body,
        out_type=jax.ShapeDtypeStruct((NC, N1, DT), jnp.float32),
        mesh=mesh,
        scratch_types=[
            pltpu.VMEM((2, CH), jnp.int32),
            pltpu.VMEM((2, CH), jnp.int32),
            pltpu.VMEM((CH, DT), jnp.float32),
            pltpu.VMEM((CH, DT), jnp.float32),
            pltpu.VMEM((CH, L), jnp.float32),
            pltpu.VMEM((CH, L), jnp.float32),
            pltpu.VMEM((UNROLL * L,), jnp.float32),
            pltpu.VMEM((CH, DT), jnp.float32),
            pltpu.VMEM((CH, DT), jnp.float32),
            pltpu.VMEM_SHARED((N1, DT), jnp.float32),
            pltpu.SemaphoreType.DMA,
            pltpu.SemaphoreType.DMA,
            pltpu.SemaphoreType.DMA,
        ],
        compiler_params=pltpu.CompilerParams(
            use_tc_tiling_on_sc=False, needs_layout_passes=False),
    )


_sc_layer1 = _make_sc_layer(D1)
_sc_layer2 = _make_sc_layer(D2)


def _dup_att_matrix(att_vec, d_in):
    k = jnp.arange(d_in)[:, None]
    l = jnp.arange(L)[None, :]
    if d_in == D1:
        mask = (k // C1) == (l % H1)
    else:
        mask = jnp.ones((d_in, L), bool)
    return att_vec[:, None] * mask.astype(jnp.float32)


def kernel(x, edge_index, W1, att_src1, att_dst1, b1, W2, att_src2, att_dst2, b2):
    ei = edge_index.astype(jnp.int32)
    loops = jnp.arange(N, dtype=jnp.int32)
    padi = jnp.full((EPAD - EL,), TRASH, jnp.int32)
    src = jnp.concatenate([ei[0], loops, padi])
    dst = jnp.concatenate([ei[1], loops, padi])
    ei3 = jnp.stack([src.reshape(N_CHUNKS, CH), dst.reshape(N_CHUNKS, CH)],
                    axis=1)

    xp = jnp.pad(x, ((0, N1 - N), (0, 0)))
    ms1 = _dup_att_matrix(att_src1.reshape(D1), D1)
    md1 = _dup_att_matrix(att_dst1.reshape(D1), D1)
    w2p = jnp.pad(W2, ((0, 0), (0, D2 - NUM_CLASSES)))
    a2s = jnp.pad(att_src2.reshape(NUM_CLASSES), (0, D2 - NUM_CLASSES))
    a2d = jnp.pad(att_dst2.reshape(NUM_CLASSES), (0, D2 - NUM_CLASSES))
    ms2 = _dup_att_matrix(a2s, D2)
    md2 = _dup_att_matrix(a2d, D2)
    rsel = (jnp.arange(D1)[None, :] // C1 == jnp.arange(H1)[:, None])
    rsel = rsel.astype(jnp.float32)

    has1, ad1 = pl.pallas_call(
        _prep1_body,
        out_shape=[
            jax.ShapeDtypeStruct((N1, D1 + L), jnp.float32),
            jax.ShapeDtypeStruct((N1, L), jnp.float32),
        ],
    )(xp, W1, ms1, md1)

    acc1 = _sc_layer1(ei3, has1, ad1)

    has2, ad2 = pl.pallas_call(
        _prep2_body,
        out_shape=[
            jax.ShapeDtypeStruct((N1, D2 + L), jnp.float32),
            jax.ShapeDtypeStruct((N1, L), jnp.float32),
        ],
    )(acc1, b1.reshape(1, D1), rsel, w2p, ms2, md2)

    acc2 = _sc_layer2(ei3, has2, ad2)

    lsm, out = pl.pallas_call(
        _final_body,
        out_shape=[
            jax.ShapeDtypeStruct((N, NUM_CLASSES), jnp.float32),
            jax.ShapeDtypeStruct((N, NUM_CLASSES), jnp.float32),
        ],
    )(acc2, b2.reshape(1, NUM_CLASSES))

    return (lsm, out)


# speedup vs baseline: 60.8907x; 1.0692x over previous
import jax
import jax.numpy as jnp
from jax import lax
from jax.experimental import pallas as pl
from jax.experimental.pallas import tpu as pltpu
from jax.experimental.pallas import tpu_sc as plsc

N = 10000
E = 320000
EL = E + N
F_IN = 128
H1, C1 = 8, 8
D1 = H1 * C1
NUM_CLASSES = 40
D2 = 48

NC, NS, L = 2, 16, 16
CH = 128
N1 = 10240
TRASH = N
N_CHUNKS = 2592
EPAD = N_CHUNKS * CH
CHUNKS_W = N_CHUNKS // (NS * NC)
ROWS_PER_TILE = N1 // NS
UNROLL = 4


def _prep1_body(x_ref, w1_ref, ms_ref, md_ref, has_ref, ad_ref):
    h = jnp.dot(x_ref[...], w1_ref[...], preferred_element_type=jnp.float32)
    hs = jnp.dot(h, ms_ref[...], preferred_element_type=jnp.float32)
    has_ref[...] = jnp.concatenate([h, hs], axis=1)
    ad_ref[...] = jnp.dot(h, md_ref[...], preferred_element_type=jnp.float32)


def _prep2_body(acc_ref, b1_ref, rsel_ref, w2_ref, ms_ref, md_ref,
                has_ref, ad_ref):
    o = acc_ref[0] + acc_ref[1]
    num = o[:, :D1]
    srec = 1.0 / (o[:, D1:D1 + H1] + 1e-16)
    sexp = jnp.dot(srec, rsel_ref[...], preferred_element_type=jnp.float32)
    o1 = num * sexp + b1_ref[...]
    x2 = jnp.where(o1 > 0, o1, jnp.exp(o1) - 1.0)
    h2 = jnp.dot(x2, w2_ref[...], preferred_element_type=jnp.float32)
    hs = jnp.dot(h2, ms_ref[...], preferred_element_type=jnp.float32)
    has_ref[...] = jnp.concatenate([h2, hs], axis=1)
    ad_ref[...] = jnp.dot(h2, md_ref[...], preferred_element_type=jnp.float32)


def _final_body(acc_ref, b2_ref, lsm_ref, out_ref):
    o = acc_ref[0] + acc_ref[1]
    num = o[:N, :NUM_CLASSES]
    sden = o[:N, D2:D2 + 1]
    res = num / (sden + 1e-16) + b2_ref[...]
    m = jnp.max(res, axis=1, keepdims=True)
    ex = jnp.exp(res - m)
    ssum = jnp.sum(ex, axis=1, keepdims=True)
    lsm_ref[...] = res - m - jnp.log(ssum)
    out_ref[...] = res


def _make_sc_layer(DH):
    DT = DH + L
    mesh = plsc.VectorSubcoreMesh(core_axis_name="c", subcore_axis_name="s")
    nj = DH // L

    assert CHUNKS_W % 3 == 0

    def body(ei_hbm, has_hbm, ad_hbm, outp,
             idx0, idx1, idx2, has0, has1, has2, ad0, ad1, ad2, alf,
             msg0, msg1, msg2, acc_sh,
             semg0, semg1, semg2, sems0, sems1, sems2):
        c = lax.axis_index("c")
        s = lax.axis_index("s")
        wid = c * NS + s
        zero16 = jnp.zeros((L,), jnp.float32)
        idxv = [idx0, idx1, idx2]
        hasv = [has0, has1, has2]
        adv = [ad0, ad1, ad2]
        msgv = [msg0, msg1, msg2]
        semg = [semg0, semg1, semg2]
        sems = [sems0, sems1, sems2]

        def _zmsg(i, _):
            for j in range(DT // L):
                msg0[i, pl.ds(j * L, L)] = zero16
            return 0
        lax.fori_loop(0, CH, _zmsg, 0)

        for t in range(ROWS_PER_TILE // CH):
            r = s * ROWS_PER_TILE + t * CH
            pltpu.sync_copy(msg0, acc_sh.at[pl.ds(r, CH)])
        plsc.subcore_barrier()

        def start_load(b, k):
            pltpu.sync_copy(ei_hbm.at[wid * CHUNKS_W + k], idxv[b])
            pltpu.async_copy(has_hbm.at[idxv[b].at[0]], hasv[b], semg[b])
            pltpu.async_copy(ad_hbm.at[idxv[b].at[1]], adv[b], semg[b])

        def wait_load(b):
            pltpu.make_async_copy(has_hbm.at[idxv[b].at[0]], hasv[b],
                                  semg[b]).wait()
            pltpu.make_async_copy(ad_hbm.at[idxv[b].at[1]], adv[b],
                                  semg[b]).wait()

        def start_scatter(b):
            pltpu.async_copy(msgv[b], acc_sh.at[idxv[b].at[1]], sems[b],
                             add=True)

        def wait_scatter(b):
            pltpu.make_async_copy(msgv[b], acc_sh.at[idxv[b].at[1]],
                                  sems[b]).wait()

        def compute(b):
            def group(g, _):
                i0 = g * UNROLL
                for u in range(UNROLL):
                    i = i0 + u
                    v = hasv[b][i, pl.ds(DH, L)] + adv[b][i]
                    v = jnp.maximum(v, v * 0.2)
                    ex = jnp.exp(v)
                    alf[pl.ds(u * L, L)] = ex
                    msgv[b][i, pl.ds(DH, L)] = ex
                lane = lax.broadcasted_iota(jnp.int32, (L,), 0)
                half = lane // 8
                for u in range(UNROLL):
                    i = i0 + u
                    for j in range(nj):
                        aej = plsc.load_gather(alf, [half + (u * L + 2 * j)])
                        msgv[b][i, pl.ds(j * L, L)] = (
                            hasv[b][i, pl.ds(j * L, L)] * aej)
                return 0
            lax.fori_loop(0, CH // UNROLL, group, 0)

        start_load(0, 0)

        def triple(t, _):
            for b in range(3):
                k = 3 * t + b
                bn = (b + 1) % 3
                if b == 2:
                    wait_scatter(bn)
                else:
                    @pl.when(t > 0)
                    def _():
                        wait_scatter(bn)
                if b == 2:
                    @pl.when(t < CHUNKS_W // 3 - 1)
                    def _():
                        start_load(bn, k + 1)
                else:
                    start_load(bn, k + 1)
                wait_load(b)
                compute(b)
                start_scatter(b)
            return 0
        lax.fori_loop(0, CHUNKS_W // 3, triple, 0)
        wait_scatter((CHUNKS_W - 2) % 3)
        wait_scatter((CHUNKS_W - 1) % 3)
        plsc.subcore_barrier()

        for t in range(ROWS_PER_TILE // CH):
            r = s * ROWS_PER_TILE + t * CH
            pltpu.sync_copy(acc_sh.at[pl.ds(r, CH)], outp.at[c].at[pl.ds(r, CH)])

    return pl.kernel(
        body,
        out_type=jax.ShapeDtypeStruct((NC, N1, DT), jnp.float32),
        mesh=mesh,
        scratch_types=(
            [pltpu.VMEM((2, CH), jnp.int32)] * 3
            + [pltpu.VMEM((CH, DT), jnp.float32)] * 3
            + [pltpu.VMEM((CH, L), jnp.float32)] * 3
            + [pltpu.VMEM((UNROLL * L,), jnp.float32)]
            + [pltpu.VMEM((CH, DT), jnp.float32)] * 3
            + [pltpu.VMEM_SHARED((N1, DT), jnp.float32)]
            + [pltpu.SemaphoreType.DMA] * 6
        ),
        compiler_params=pltpu.CompilerParams(
            use_tc_tiling_on_sc=False, needs_layout_passes=False),
    )


_sc_layer1 = _make_sc_layer(D1)
_sc_layer2 = _make_sc_layer(D2)


def _dup_att_matrix(att_vec, d_in):
    k = jnp.arange(d_in)[:, None]
    l = jnp.arange(L)[None, :]
    if d_in == D1:
        mask = (k // C1) == (l % H1)
    else:
        mask = jnp.ones((d_in, L), bool)
    return att_vec[:, None] * mask.astype(jnp.float32)


def kernel(x, edge_index, W1, att_src1, att_dst1, b1, W2, att_src2, att_dst2, b2):
    ei = edge_index.astype(jnp.int32)
    loops = jnp.arange(N, dtype=jnp.int32)
    padi = jnp.full((EPAD - EL,), TRASH, jnp.int32)
    src = jnp.concatenate([ei[0], loops, padi])
    dst = jnp.concatenate([ei[1], loops, padi])
    ei3 = jnp.stack([src.reshape(N_CHUNKS, CH), dst.reshape(N_CHUNKS, CH)],
                    axis=1)

    xp = jnp.pad(x, ((0, N1 - N), (0, 0)))
    ms1 = _dup_att_matrix(att_src1.reshape(D1), D1)
    md1 = _dup_att_matrix(att_dst1.reshape(D1), D1)
    w2p = jnp.pad(W2, ((0, 0), (0, D2 - NUM_CLASSES)))
    a2s = jnp.pad(att_src2.reshape(NUM_CLASSES), (0, D2 - NUM_CLASSES))
    a2d = jnp.pad(att_dst2.reshape(NUM_CLASSES), (0, D2 - NUM_CLASSES))
    ms2 = _dup_att_matrix(a2s, D2)
    md2 = _dup_att_matrix(a2d, D2)
    rsel = (jnp.arange(D1)[None, :] // C1 == jnp.arange(H1)[:, None])
    rsel = rsel.astype(jnp.float32)

    has1, ad1 = pl.pallas_call(
        _prep1_body,
        out_shape=[
            jax.ShapeDtypeStruct((N1, D1 + L), jnp.float32),
            jax.ShapeDtypeStruct((N1, L), jnp.float32),
        ],
    )(xp, W1, ms1, md1)

    acc1 = _sc_layer1(ei3, has1, ad1)

    has2, ad2 = pl.pallas_call(
        _prep2_body,
        out_shape=[
            jax.ShapeDtypeStruct((N1, D2 + L), jnp.float32),
            jax.ShapeDtypeStruct((N1, L), jnp.float32),
        ],
    )(acc1, b1.reshape(1, D1), rsel, w2p, ms2, md2)

    acc2 = _sc_layer2(ei3, has2, ad2)

    lsm, out = pl.pallas_call(
        _final_body,
        out_shape=[
            jax.ShapeDtypeStruct((N, NUM_CLASSES), jnp.float32),
            jax.ShapeDtypeStruct((N, NUM_CLASSES), jnp.float32),
        ],
    )(acc2, b2.reshape(1, NUM_CLASSES))

    return (lsm, out)


# speedup vs baseline: 64.8770x; 1.0655x over previous
import jax
import jax.numpy as jnp
from jax import lax
from jax.experimental import pallas as pl
from jax.experimental.pallas import tpu as pltpu
from jax.experimental.pallas import tpu_sc as plsc

N = 10000
E = 320000
EL = E + N
F_IN = 128
H1, C1 = 8, 8
D1 = H1 * C1
NUM_CLASSES = 40
D2 = 48

NC, NS, L = 2, 16, 16
CH = 128
N1 = 10240
TRASH = N
N_CHUNKS = 2592
EPAD = N_CHUNKS * CH
CHUNKS_W = N_CHUNKS // (NS * NC)
ROWS_PER_TILE = N1 // NS
UNROLL = 4


def _prep1_body(x_ref, w1_ref, ms_ref, md_ref, has_ref, ad_ref):
    h = jnp.dot(x_ref[...], w1_ref[...], preferred_element_type=jnp.float32)
    hs = jnp.dot(h, ms_ref[...], preferred_element_type=jnp.float32)
    has_ref[...] = jnp.concatenate([h, hs], axis=1)
    ad_ref[...] = jnp.dot(h, md_ref[...], preferred_element_type=jnp.float32)


def _prep2_body(acc_ref, b1_ref, rsel_ref, w2_ref, ms_ref, md_ref,
                has_ref, ad_ref):
    o = acc_ref[0] + acc_ref[1]
    num = o[:, :D1]
    srec = 1.0 / (o[:, D1:D1 + H1] + 1e-16)
    sexp = jnp.dot(srec, rsel_ref[...], preferred_element_type=jnp.float32)
    o1 = num * sexp + b1_ref[...]
    x2 = jnp.where(o1 > 0, o1, jnp.exp(o1) - 1.0)
    h2 = jnp.dot(x2, w2_ref[...], preferred_element_type=jnp.float32)
    ones_col = (lax.broadcasted_iota(jnp.int32, (1, D2), 1)
                == NUM_CLASSES).astype(jnp.float32)
    h2 = h2 + ones_col
    hs = jnp.dot(h2, ms_ref[...], preferred_element_type=jnp.float32)
    has_ref[...] = jnp.concatenate([h2, hs], axis=1)
    ad_ref[...] = jnp.dot(h2, md_ref[...], preferred_element_type=jnp.float32)


def _final_body(acc_ref, b2_ref, lsm_ref, out_ref):
    o = acc_ref[0] + acc_ref[1]
    num = o[:N, :NUM_CLASSES]
    sden = o[:N, NUM_CLASSES:NUM_CLASSES + 1]
    res = num / (sden + 1e-16) + b2_ref[...]
    m = jnp.max(res, axis=1, keepdims=True)
    ex = jnp.exp(res - m)
    ssum = jnp.sum(ex, axis=1, keepdims=True)
    lsm_ref[...] = res - m - jnp.log(ssum)
    out_ref[...] = res


def _make_sc_layer(DH, multi_head):
    GW = DH + L
    DT = DH + L if multi_head else DH
    mesh = plsc.VectorSubcoreMesh(core_axis_name="c", subcore_axis_name="s")
    nj = DH // L

    assert CHUNKS_W % 3 == 0

    def body(ei_hbm, has_hbm, ad_hbm, outp,
             idx0, idx1, idx2, has0, has1, has2, ad0, ad1, ad2, alf,
             msg0, msg1, msg2, acc_sh,
             semg0, semg1, semg2, sems0, sems1, sems2):
        c = lax.axis_index("c")
        s = lax.axis_index("s")
        wid = c * NS + s
        zero16 = jnp.zeros((L,), jnp.float32)
        idxv = [idx0, idx1, idx2]
        hasv = [has0, has1, has2]
        adv = [ad0, ad1, ad2]
        msgv = [msg0, msg1, msg2]
        semg = [semg0, semg1, semg2]
        sems = [sems0, sems1, sems2]

        def _zmsg(i, _):
            for j in range(DT // L):
                msg0[i, pl.ds(j * L, L)] = zero16
            return 0
        lax.fori_loop(0, CH, _zmsg, 0)

        for t in range(ROWS_PER_TILE // CH):
            r = s * ROWS_PER_TILE + t * CH
            pltpu.sync_copy(msg0, acc_sh.at[pl.ds(r, CH)])
        plsc.subcore_barrier()

        def start_load(b, k):
            pltpu.sync_copy(ei_hbm.at[wid * CHUNKS_W + k], idxv[b])
            pltpu.async_copy(has_hbm.at[idxv[b].at[0]], hasv[b], semg[b])
            pltpu.async_copy(ad_hbm.at[idxv[b].at[1]], adv[b], semg[b])

        def wait_load(b):
            pltpu.make_async_copy(has_hbm.at[idxv[b].at[0]], hasv[b],
                                  semg[b]).wait()
            pltpu.make_async_copy(ad_hbm.at[idxv[b].at[1]], adv[b],
                                  semg[b]).wait()

        def start_scatter(b):
            pltpu.async_copy(msgv[b], acc_sh.at[idxv[b].at[1]], sems[b],
                             add=True)

        def wait_scatter(b):
            pltpu.make_async_copy(msgv[b], acc_sh.at[idxv[b].at[1]],
                                  sems[b]).wait()

        def compute(b):
            def group(g, _):
                i0 = g * UNROLL
                if multi_head:
                    for u in range(UNROLL):
                        i = i0 + u
                        v = hasv[b][i, pl.ds(DH, L)] + adv[b][i]
                        v = jnp.maximum(v, v * 0.2)
                        ex = jnp.exp(v)
                        alf[pl.ds(u * L, L)] = ex
                        msgv[b][i, pl.ds(DH, L)] = ex
                    lane = lax.broadcasted_iota(jnp.int32, (L,), 0)
                    half = lane // 8
                    for u in range(UNROLL):
                        i = i0 + u
                        for j in range(nj):
                            aej = plsc.load_gather(
                                alf, [half + (u * L + 2 * j)])
                            msgv[b][i, pl.ds(j * L, L)] = (
                                hasv[b][i, pl.ds(j * L, L)] * aej)
                else:
                    for u in range(UNROLL):
                        i = i0 + u
                        v = hasv[b][i, pl.ds(DH, L)] + adv[b][i]
                        v = jnp.maximum(v, v * 0.2)
                        ex = jnp.exp(v)
                        for j in range(nj):
                            msgv[b][i, pl.ds(j * L, L)] = (
                                hasv[b][i, pl.ds(j * L, L)] * ex)
                return 0
            lax.fori_loop(0, CH // UNROLL, group, 0)

        start_load(0, 0)

        def triple(t, _):
            for b in range(3):
                k = 3 * t + b
                bn = (b + 1) % 3
                if b == 2:
                    wait_scatter(bn)
                else:
                    @pl.when(t > 0)
                    def _():
                        wait_scatter(bn)
                if b == 2:
                    @pl.when(t < CHUNKS_W // 3 - 1)
                    def _():
                        start_load(bn, k + 1)
                else:
                    start_load(bn, k + 1)
                wait_load(b)
                compute(b)
                start_scatter(b)
            return 0
        lax.fori_loop(0, CHUNKS_W // 3, triple, 0)
        wait_scatter((CHUNKS_W - 2) % 3)
        wait_scatter((CHUNKS_W - 1) % 3)
        plsc.subcore_barrier()

        for t in range(ROWS_PER_TILE // CH):
            r = s * ROWS_PER_TILE + t * CH
            pltpu.sync_copy(acc_sh.at[pl.ds(r, CH)], outp.at[c].at[pl.ds(r, CH)])

    return pl.kernel(
        body,
        out_type=jax.ShapeDtypeStruct((NC, N1, DT), jnp.float32),
        mesh=mesh,
        scratch_types=(
            [pltpu.VMEM((2, CH), jnp.int32)] * 3
            + [pltpu.VMEM((CH, GW), jnp.float32)] * 3
            + [pltpu.VMEM((CH, L), jnp.float32)] * 3
            + [pltpu.VMEM((UNROLL * L,), jnp.float32)]
            + [pltpu.VMEM((CH, DT), jnp.float32)] * 3
            + [pltpu.VMEM_SHARED((N1, DT), jnp.float32)]
            + [pltpu.SemaphoreType.DMA] * 6
        ),
        compiler_params=pltpu.CompilerParams(
            use_tc_tiling_on_sc=False, needs_layout_passes=False),
    )


_sc_layer1 = _make_sc_layer(D1, True)
_sc_layer2 = _make_sc_layer(D2, False)


def _dup_att_matrix(att_vec, d_in):
    k = jnp.arange(d_in)[:, None]
    l = jnp.arange(L)[None, :]
    if d_in == D1:
        mask = (k // C1) == (l % H1)
    else:
        mask = jnp.ones((d_in, L), bool)
    return att_vec[:, None] * mask.astype(jnp.float32)


def kernel(x, edge_index, W1, att_src1, att_dst1, b1, W2, att_src2, att_dst2, b2):
    ei = edge_index.astype(jnp.int32)
    loops = jnp.arange(N, dtype=jnp.int32)
    padi = jnp.full((EPAD - EL,), TRASH, jnp.int32)
    src = jnp.concatenate([ei[0], loops, padi])
    dst = jnp.concatenate([ei[1], loops, padi])
    ei3 = jnp.stack([src.reshape(N_CHUNKS, CH), dst.reshape(N_CHUNKS, CH)],
                    axis=1)

    xp = jnp.pad(x, ((0, N1 - N), (0, 0)))
    ms1 = _dup_att_matrix(att_src1.reshape(D1), D1)
    md1 = _dup_att_matrix(att_dst1.reshape(D1), D1)
    w2p = jnp.pad(W2, ((0, 0), (0, D2 - NUM_CLASSES)))
    a2s = jnp.pad(att_src2.reshape(NUM_CLASSES), (0, D2 - NUM_CLASSES))
    a2d = jnp.pad(att_dst2.reshape(NUM_CLASSES), (0, D2 - NUM_CLASSES))
    ms2 = _dup_att_matrix(a2s, D2)
    md2 = _dup_att_matrix(a2d, D2)
    rsel = (jnp.arange(D1)[None, :] // C1 == jnp.arange(H1)[:, None])
    rsel = rsel.astype(jnp.float32)

    has1, ad1 = pl.pallas_call(
        _prep1_body,
        out_shape=[
            jax.ShapeDtypeStruct((N1, D1 + L), jnp.float32),
            jax.ShapeDtypeStruct((N1, L), jnp.float32),
        ],
    )(xp, W1, ms1, md1)

    acc1 = _sc_layer1(ei3, has1, ad1)

    has2, ad2 = pl.pallas_call(
        _prep2_body,
        out_shape=[
            jax.ShapeDtypeStruct((N1, D2 + L), jnp.float32),
            jax.ShapeDtypeStruct((N1, L), jnp.float32),
        ],
    )(acc1, b1.reshape(1, D1), rsel, w2p, ms2, md2)

    acc2 = _sc_layer2(ei3, has2, ad2)

    lsm, out = pl.pallas_call(
        _final_body,
        out_shape=[
            jax.ShapeDtypeStruct((N, NUM_CLASSES), jnp.float32),
            jax.ShapeDtypeStruct((N, NUM_CLASSES), jnp.float32),
        ],
    )(acc2, b2.reshape(1, NUM_CLASSES))

    return (lsm, out)


# speedup vs baseline: 65.0350x; 1.0024x over previous
import jax
import jax.numpy as jnp
from jax import lax
from jax.experimental import pallas as pl
from jax.experimental.pallas import tpu as pltpu
from jax.experimental.pallas import tpu_sc as plsc

N = 10000
E = 320000
EL = E + N
F_IN = 128
H1, C1 = 8, 8
D1 = H1 * C1
NUM_CLASSES = 40
D2 = 48

NC, NS, L = 2, 16, 16
CH = 128
N1 = 10016
TRASH = N
N_CHUNKS = 2592
EPAD = N_CHUNKS * CH
CHUNKS_W = N_CHUNKS // (NS * NC)
ROWS_PER_TILE = N1 // NS
UNROLL = 4


def _prep1_body(x_ref, w1_ref, ms_ref, md_ref, has_ref, ad_ref):
    h = jnp.dot(x_ref[...], w1_ref[...], preferred_element_type=jnp.float32)
    hs = jnp.dot(h, ms_ref[...], preferred_element_type=jnp.float32)
    has_ref[...] = jnp.concatenate([h, hs], axis=1)
    ad_ref[...] = jnp.dot(h, md_ref[...], preferred_element_type=jnp.float32)


def _prep2_body(acc_ref, b1_ref, rsel_ref, w2_ref, ms_ref, md_ref,
                has_ref, ad_ref):
    o = acc_ref[0] + acc_ref[1]
    num = o[:, :D1]
    srec = 1.0 / (o[:, D1:D1 + H1] + 1e-16)
    sexp = jnp.dot(srec, rsel_ref[...], preferred_element_type=jnp.float32)
    o1 = num * sexp + b1_ref[...]
    x2 = jnp.where(o1 > 0, o1, jnp.exp(o1) - 1.0)
    h2 = jnp.dot(x2, w2_ref[...], preferred_element_type=jnp.float32)
    ones_col = (lax.broadcasted_iota(jnp.int32, (1, D2), 1)
                == NUM_CLASSES).astype(jnp.float32)
    h2 = h2 + ones_col
    hs = jnp.dot(h2, ms_ref[...], preferred_element_type=jnp.float32)
    has_ref[...] = jnp.concatenate([h2, hs], axis=1)
    ad_ref[...] = jnp.dot(h2, md_ref[...], preferred_element_type=jnp.float32)


def _final_body(acc_ref, b2_ref, lsm_ref, out_ref):
    o = acc_ref[0] + acc_ref[1]
    num = o[:N, :NUM_CLASSES]
    sden = o[:N, NUM_CLASSES:NUM_CLASSES + 1]
    res = num / (sden + 1e-16) + b2_ref[...]
    m = jnp.max(res, axis=1, keepdims=True)
    ex = jnp.exp(res - m)
    ssum = jnp.sum(ex, axis=1, keepdims=True)
    lsm_ref[...] = res - m - jnp.log(ssum)
    out_ref[...] = res


def _make_sc_layer(DH, multi_head):
    GW = DH + L
    DT = DH + L if multi_head else DH
    mesh = plsc.VectorSubcoreMesh(core_axis_name="c", subcore_axis_name="s")
    nj = DH // L

    assert CHUNKS_W % 3 == 0

    def body(ei_hbm, has_hbm, ad_hbm, outp,
             idx0, idx1, idx2, has0, has1, has2, ad0, ad1, ad2, alf,
             msg0, msg1, msg2, acc_sh,
             semg0, semg1, semg2, sems0, sems1, sems2):
        c = lax.axis_index("c")
        s = lax.axis_index("s")
        wid = c * NS + s
        zero16 = jnp.zeros((L,), jnp.float32)
        idxv = [idx0, idx1, idx2]
        hasv = [has0, has1, has2]
        adv = [ad0, ad1, ad2]
        msgv = [msg0, msg1, msg2]
        semg = [semg0, semg1, semg2]
        sems = [sems0, sems1, sems2]

        pltpu.sync_copy(ei_hbm.at[wid * CHUNKS_W], idx0)
        pltpu.async_copy(has_hbm.at[idx0.at[0]], has0, semg0)
        pltpu.async_copy(ad_hbm.at[idx0.at[1]], ad0, semg0)

        def _zmsg(i, _):
            for j in range(DT // L):
                msg0[i, pl.ds(j * L, L)] = zero16
            return 0
        lax.fori_loop(0, CH, _zmsg, 0)

        for t in range(-(-ROWS_PER_TILE // CH)):
            r = s * ROWS_PER_TILE + t * CH
            rows = min(CH, ROWS_PER_TILE - t * CH)
            pltpu.sync_copy(msg0.at[pl.ds(0, rows)],
                            acc_sh.at[pl.ds(r, rows)])
        plsc.subcore_barrier()

        def start_load(b, k):
            pltpu.sync_copy(ei_hbm.at[wid * CHUNKS_W + k], idxv[b])
            pltpu.async_copy(has_hbm.at[idxv[b].at[0]], hasv[b], semg[b])
            pltpu.async_copy(ad_hbm.at[idxv[b].at[1]], adv[b], semg[b])

        def wait_load(b):
            pltpu.make_async_copy(has_hbm.at[idxv[b].at[0]], hasv[b],
                                  semg[b]).wait()
            pltpu.make_async_copy(ad_hbm.at[idxv[b].at[1]], adv[b],
                                  semg[b]).wait()

        def start_scatter(b):
            pltpu.async_copy(msgv[b], acc_sh.at[idxv[b].at[1]], sems[b],
                             add=True)

        def wait_scatter(b):
            pltpu.make_async_copy(msgv[b], acc_sh.at[idxv[b].at[1]],
                                  sems[b]).wait()

        def compute(b):
            def group(g, _):
                i0 = g * UNROLL
                if multi_head:
                    for u in range(UNROLL):
                        i = i0 + u
                        v = hasv[b][i, pl.ds(DH, L)] + adv[b][i]
                        v = jnp.maximum(v, v * 0.2)
                        ex = jnp.exp(v)
                        alf[pl.ds(u * L, L)] = ex
                        msgv[b][i, pl.ds(DH, L)] = ex
                    lane = lax.broadcasted_iota(jnp.int32, (L,), 0)
                    half = lane // 8
                    for u in range(UNROLL):
                        i = i0 + u
                        for j in range(nj):
                            aej = plsc.load_gather(
                                alf, [half + (u * L + 2 * j)])
                            msgv[b][i, pl.ds(j * L, L)] = (
                                hasv[b][i, pl.ds(j * L, L)] * aej)
                else:
                    for u in range(UNROLL):
                        i = i0 + u
                        v = hasv[b][i, pl.ds(DH, L)] + adv[b][i]
                        v = jnp.maximum(v, v * 0.2)
                        ex = jnp.exp(v)
                        for j in range(nj):
                            msgv[b][i, pl.ds(j * L, L)] = (
                                hasv[b][i, pl.ds(j * L, L)] * ex)
                return 0
            lax.fori_loop(0, CH // UNROLL, group, 0)

        def triple(t, _):
            for b in range(3):
                k = 3 * t + b
                bn = (b + 1) % 3
                if b == 2:
                    wait_scatter(bn)
                else:
                    @pl.when(t > 0)
                    def _():
                        wait_scatter(bn)
                if b == 2:
                    @pl.when(t < CHUNKS_W // 3 - 1)
                    def _():
                        start_load(bn, k + 1)
                else:
                    start_load(bn, k + 1)
                wait_load(b)
                compute(b)
                start_scatter(b)
            return 0
        lax.fori_loop(0, CHUNKS_W // 3, triple, 0)
        wait_scatter((CHUNKS_W - 2) % 3)
        wait_scatter((CHUNKS_W - 1) % 3)
        plsc.subcore_barrier()

        for t in range(-(-ROWS_PER_TILE // CH)):
            r = s * ROWS_PER_TILE + t * CH
            rows = min(CH, ROWS_PER_TILE - t * CH)
            pltpu.sync_copy(acc_sh.at[pl.ds(r, rows)],
                            outp.at[c].at[pl.ds(r, rows)])

    return pl.kernel(
        body,
        out_type=jax.ShapeDtypeStruct((NC, N1, DT), jnp.float32),
        mesh=mesh,
        scratch_types=(
            [pltpu.VMEM((2, CH), jnp.int32)] * 3
            + [pltpu.VMEM((CH, GW), jnp.float32)] * 3
            + [pltpu.VMEM((CH, L), jnp.float32)] * 3
            + [pltpu.VMEM((UNROLL * L,), jnp.float32)]
            + [pltpu.VMEM((CH, DT), jnp.float32)] * 3
            + [pltpu.VMEM_SHARED((N1, DT), jnp.float32)]
            + [pltpu.SemaphoreType.DMA] * 6
        ),
        compiler_params=pltpu.CompilerParams(
            use_tc_tiling_on_sc=False, needs_layout_passes=False),
    )


_sc_layer1 = _make_sc_layer(D1, True)
_sc_layer2 = _make_sc_layer(D2, False)


def _dup_att_matrix(att_vec, d_in):
    k = jnp.arange(d_in)[:, None]
    l = jnp.arange(L)[None, :]
    if d_in == D1:
        mask = (k // C1) == (l % H1)
    else:
        mask = jnp.ones((d_in, L), bool)
    return att_vec[:, None] * mask.astype(jnp.float32)


def kernel(x, edge_index, W1, att_src1, att_dst1, b1, W2, att_src2, att_dst2, b2):
    ei = edge_index.astype(jnp.int32)
    loops = jnp.arange(N, dtype=jnp.int32)
    padi = jnp.full((EPAD - EL,), TRASH, jnp.int32)
    src = jnp.concatenate([ei[0], loops, padi])
    dst = jnp.concatenate([ei[1], loops, padi])
    ei3 = jnp.stack([src.reshape(N_CHUNKS, CH), dst.reshape(N_CHUNKS, CH)],
                    axis=1)

    xp = jnp.pad(x, ((0, N1 - N), (0, 0)))
    ms1 = _dup_att_matrix(att_src1.reshape(D1), D1)
    md1 = _dup_att_matrix(att_dst1.reshape(D1), D1)
    w2p = jnp.pad(W2, ((0, 0), (0, D2 - NUM_CLASSES)))
    a2s = jnp.pad(att_src2.reshape(NUM_CLASSES), (0, D2 - NUM_CLASSES))
    a2d = jnp.pad(att_dst2.reshape(NUM_CLASSES), (0, D2 - NUM_CLASSES))
    ms2 = _dup_att_matrix(a2s, D2)
    md2 = _dup_att_matrix(a2d, D2)
    rsel = (jnp.arange(D1)[None, :] // C1 == jnp.arange(H1)[:, None])
    rsel = rsel.astype(jnp.float32)

    has1, ad1 = pl.pallas_call(
        _prep1_body,
        out_shape=[
            jax.ShapeDtypeStruct((N1, D1 + L), jnp.float32),
            jax.ShapeDtypeStruct((N1, L), jnp.float32),
        ],
    )(xp, W1, ms1, md1)

    acc1 = _sc_layer1(ei3, has1, ad1)

    has2, ad2 = pl.pallas_call(
        _prep2_body,
        out_shape=[
            jax.ShapeDtypeStruct((N1, D2 + L), jnp.float32),
            jax.ShapeDtypeStruct((N1, L), jnp.float32),
        ],
    )(acc1, b1.reshape(1, D1), rsel, w2p, ms2, md2)

    acc2 = _sc_layer2(ei3, has2, ad2)

    lsm, out = pl.pallas_call(
        _final_body,
        out_shape=[
            jax.ShapeDtypeStruct((N, NUM_CLASSES), jnp.float32),
            jax.ShapeDtypeStruct((N, NUM_CLASSES), jnp.float32),
        ],
    )(acc2, b2.reshape(1, NUM_CLASSES))

    return (lsm, out)
